# Initial kernel scaffold; baseline (speedup 1.0000x reference)
import jax
import jax.numpy as jnp
from jax.experimental import pallas as pl


def kernel(x, edge_index, edge_attr, W, b):
    raise NotImplementedError("write your pallas kernel here")


# speedup vs baseline: 3.4604x; 3.4604x over previous
import functools

import jax
import jax.numpy as jnp
from jax import lax
from jax.experimental import pallas as pl
from jax.experimental.pallas import tpu as pltpu
from jax.experimental.pallas import tpu_sc as plsc

N, E, D, DE, DOUT = 10000, 320000, 128, 16, 128

NC, NS, L = 2, 16, 16
NW = NC * NS
EW = E // NW
C = 80
NCHUNK = EW // C
VPR = DOUT // L


def _node_mm_body(x_ref, w1_ref, w2_ref, y1_ref, y2_ref):
    xv = x_ref[...]
    y1_ref[...] = jnp.dot(xv, w1_ref[...], preferred_element_type=jnp.float32)
    y2_ref[...] = jnp.dot(xv, w2_ref[...], preferred_element_type=jnp.float32)


_node_mm = pl.pallas_call(
    _node_mm_body,
    out_shape=[
        jax.ShapeDtypeStruct((N, DOUT), jnp.float32),
        jax.ShapeDtypeStruct((N, DOUT), jnp.float32),
    ],
)


def _sc_gather_body(y1_hbm, y2_hbm, row_hbm, col_hbm, out_hbm,
                    row_v, col_v, g1, g2, sem1, sem2):
    wid = lax.axis_index("s") * NC + lax.axis_index("c")
    base = wid * EW
    pltpu.sync_copy(row_hbm.at[pl.ds(base, EW)], row_v)
    pltpu.sync_copy(col_hbm.at[pl.ds(base, EW)], col_v)

    def chunk_body(c, carry):
        off = c * C
        cp1 = pltpu.async_copy(y1_hbm.at[row_v.at[pl.ds(off, C)]], g1, sem1)
        cp2 = pltpu.async_copy(y2_hbm.at[col_v.at[pl.ds(off, C)]], g2, sem2)
        cp1.wait()
        cp2.wait()

        def add_body(i, carry2):
            for k in range(VPR):
                sl = pl.ds(k * L, L)
                g1[i, sl] = g1[i, sl] + g2[i, sl]
            return carry2

        lax.fori_loop(0, C, add_body, 0)
        pltpu.sync_copy(g1, out_hbm.at[pl.ds(base + off, C)])
        return carry

    lax.fori_loop(0, NCHUNK, chunk_body, 0)


_sc_gather = functools.partial(
    pl.kernel,
    out_type=jax.ShapeDtypeStruct((E, DOUT), jnp.float32),
    mesh=plsc.VectorSubcoreMesh(core_axis_name="c", subcore_axis_name="s"),
    scratch_types=[
        pltpu.VMEM((EW,), jnp.int32),
        pltpu.VMEM((EW,), jnp.int32),
        pltpu.VMEM((C, DOUT), jnp.float32),
        pltpu.VMEM((C, DOUT), jnp.float32),
        pltpu.SemaphoreType.DMA,
        pltpu.SemaphoreType.DMA,
    ],
)(_sc_gather_body)


def _edge_mm_body(s_ref, ea_ref, w3_ref, b_ref, o_ref):
    o_ref[...] = (
        s_ref[...]
        + jnp.dot(ea_ref[...], w3_ref[...], preferred_element_type=jnp.float32)
        + b_ref[...]
    )


_EB = 8000

_edge_mm = pl.pallas_call(
    _edge_mm_body,
    grid=(E // _EB,),
    in_specs=[
        pl.BlockSpec((_EB, DOUT), lambda i: (i, 0)),
        pl.BlockSpec((_EB, DE), lambda i: (i, 0)),
        pl.BlockSpec((DE, DOUT), lambda i: (0, 0)),
        pl.BlockSpec((1, DOUT), lambda i: (0, 0)),
    ],
    out_specs=pl.BlockSpec((_EB, DOUT), lambda i: (i, 0)),
    out_shape=jax.ShapeDtypeStruct((E, DOUT), jnp.float32),
)


def kernel(x, edge_index, edge_attr, W, b):
    w1 = W[:D]
    w2 = W[D:2 * D]
    w3 = W[2 * D:]
    row = edge_index[0]
    col = edge_index[1]
    y1, y2 = _node_mm(x, w1, w2)
    s = _sc_gather(y1, y2, row, col)
    return _edge_mm(s, edge_attr, w3, b.reshape(1, DOUT))


# speedup vs baseline: 4.0589x; 1.1730x over previous
import functools

import jax
import jax.numpy as jnp
from jax import lax
from jax.experimental import pallas as pl
from jax.experimental.pallas import tpu as pltpu
from jax.experimental.pallas import tpu_sc as plsc

N, E, D, DE, DOUT = 10000, 320000, 128, 16, 128

NC, NS, L = 2, 16, 16
NW = NC * NS
EW = E // NW
C = 80
NCHUNK = EW // C
NPAIR = (NCHUNK - 1) // 2
VPR = DOUT // L


def _node_mm_body(x_ref, w1_ref, w2_ref, y1_ref, y2_ref):
    xv = x_ref[...]
    y1_ref[...] = jnp.dot(xv, w1_ref[...], preferred_element_type=jnp.float32)
    y2_ref[...] = jnp.dot(xv, w2_ref[...], preferred_element_type=jnp.float32)


_node_mm = pl.pallas_call(
    _node_mm_body,
    out_shape=[
        jax.ShapeDtypeStruct((N, DOUT), jnp.float32),
        jax.ShapeDtypeStruct((N, DOUT), jnp.float32),
    ],
)


def _ea_mm_body(ea_ref, w3_ref, b_ref, o_ref):
    o_ref[...] = (
        jnp.dot(ea_ref[...], w3_ref[...], preferred_element_type=jnp.float32)
        + b_ref[...]
    )


_EB = 8000

_ea_mm = pl.pallas_call(
    _ea_mm_body,
    grid=(E // _EB,),
    in_specs=[
        pl.BlockSpec((_EB, DE), lambda i: (i, 0)),
        pl.BlockSpec((DE, DOUT), lambda i: (0, 0)),
        pl.BlockSpec((1, DOUT), lambda i: (0, 0)),
    ],
    out_specs=pl.BlockSpec((_EB, DOUT), lambda i: (i, 0)),
    out_shape=jax.ShapeDtypeStruct((E, DOUT), jnp.float32),
)


def _sc_gather_body(y1_hbm, y2_hbm, ea_hbm, row_hbm, col_hbm, out_hbm,
                    row_v, col_v,
                    a1, a2, ae, b1, b2, be, oa, ob,
                    sga, sgb, swa, swb):
    wid = lax.axis_index("s") * NC + lax.axis_index("c")
    base = wid * EW
    pltpu.sync_copy(row_hbm.at[pl.ds(base, EW)], row_v)
    pltpu.sync_copy(col_hbm.at[pl.ds(base, EW)], col_v)

    def issue_gathers(c, g1, g2, ge, sem):
        off = c * C
        pltpu.async_copy(y1_hbm.at[row_v.at[pl.ds(off, C)]], g1, sem)
        pltpu.async_copy(y2_hbm.at[col_v.at[pl.ds(off, C)]], g2, sem)
        pltpu.async_copy(ea_hbm.at[pl.ds(base + off, C)], ge, sem)

    def wait_gathers(c, g1, g2, ge, sem):
        off = c * C
        pltpu.make_async_copy(y1_hbm.at[row_v.at[pl.ds(off, C)]], g1, sem).wait()
        pltpu.make_async_copy(y2_hbm.at[col_v.at[pl.ds(off, C)]], g2, sem).wait()
        pltpu.make_async_copy(ea_hbm.at[pl.ds(base + off, C)], ge, sem).wait()

    def issue_write(c, o, sem):
        pltpu.async_copy(o, out_hbm.at[pl.ds(base + c * C, C)], sem)

    def wait_write(c, o, sem):
        pltpu.make_async_copy(o, out_hbm.at[pl.ds(base + c * C, C)], sem).wait()

    def compute(g1, g2, ge, o):
        def row_body(i, carry):
            for k in range(VPR):
                sl = pl.ds(k * L, L)
                o[i, sl] = g1[i, sl] + g2[i, sl] + ge[i, sl]
            return carry

        lax.fori_loop(0, C, row_body, 0)

    issue_gathers(0, a1, a2, ae, sga)
    issue_gathers(1, b1, b2, be, sgb)

    def pair_body(p, carry):
        c0 = 2 * p
        c1 = c0 + 1
        wait_gathers(c0, a1, a2, ae, sga)

        @pl.when(c0 >= 2)
        def _():
            wait_write(c0 - 2, oa, swa)

        compute(a1, a2, ae, oa)
        issue_write(c0, oa, swa)
        issue_gathers(c0 + 2, a1, a2, ae, sga)
        wait_gathers(c1, b1, b2, be, sgb)

        @pl.when(c1 >= 3)
        def _():
            wait_write(c1 - 2, ob, swb)

        compute(b1, b2, be, ob)
        issue_write(c1, ob, swb)

        @pl.when(c1 + 2 < NCHUNK)
        def _():
            issue_gathers(c1 + 2, b1, b2, be, sgb)

        return carry

    lax.fori_loop(0, NPAIR, pair_body, 0)

    cl = NCHUNK - 1
    wait_gathers(cl, a1, a2, ae, sga)
    wait_write(cl - 2, oa, swa)
    compute(a1, a2, ae, oa)
    issue_write(cl, oa, swa)
    wait_write(cl, oa, swa)
    wait_write(cl - 1, ob, swb)


_sc_gather = functools.partial(
    pl.kernel,
    out_type=jax.ShapeDtypeStruct((E, DOUT), jnp.float32),
    mesh=plsc.VectorSubcoreMesh(core_axis_name="c", subcore_axis_name="s"),
    scratch_types=[
        pltpu.VMEM((EW,), jnp.int32),
        pltpu.VMEM((EW,), jnp.int32),
        pltpu.VMEM((C, DOUT), jnp.float32),
        pltpu.VMEM((C, DOUT), jnp.float32),
        pltpu.VMEM((C, DOUT), jnp.float32),
        pltpu.VMEM((C, DOUT), jnp.float32),
        pltpu.VMEM((C, DOUT), jnp.float32),
        pltpu.VMEM((C, DOUT), jnp.float32),
        pltpu.VMEM((C, DOUT), jnp.float32),
        pltpu.VMEM((C, DOUT), jnp.float32),
        pltpu.SemaphoreType.DMA,
        pltpu.SemaphoreType.DMA,
        pltpu.SemaphoreType.DMA,
        pltpu.SemaphoreType.DMA,
    ],
)(_sc_gather_body)


def kernel(x, edge_index, edge_attr, W, b):
    w1 = W[:D]
    w2 = W[D:2 * D]
    w3 = W[2 * D:]
    row = edge_index[0]
    col = edge_index[1]
    y1, y2 = _node_mm(x, w1, w2)
    ea = _ea_mm(edge_attr, w3, b.reshape(1, DOUT))
    return _sc_gather(y1, y2, ea, row, col)


# speedup vs baseline: 5.4818x; 1.3505x over previous
import functools

import jax
import jax.numpy as jnp
from jax import lax
from jax.experimental import pallas as pl
from jax.experimental.pallas import tpu as pltpu
from jax.experimental.pallas import tpu_sc as plsc

N, E, D, DE, DOUT = 10000, 320000, 128, 16, 128

NC, NS, L = 2, 16, 16
NW = NC * NS
EW = E // NW
C = 80
NCHUNK = EW // C
NPAIR = (NCHUNK - 1) // 2
VPR = DOUT // L


def _node_mm_body(x_ref, w1_ref, w2_ref, y1_ref, y2_ref):
    xv = x_ref[...]
    y1_ref[...] = jnp.dot(xv, w1_ref[...], preferred_element_type=jnp.float32)
    y2_ref[...] = jnp.dot(xv, w2_ref[...], preferred_element_type=jnp.float32)


_node_mm = pl.pallas_call(
    _node_mm_body,
    out_shape=[
        jax.ShapeDtypeStruct((N, DOUT), jnp.float32),
        jax.ShapeDtypeStruct((N, DOUT), jnp.float32),
    ],
)


def _ea_mm_body(eat_ref, w3_ref, b_ref, o_ref):
    o_ref[...] = (
        jax.lax.dot_general(
            eat_ref[...], w3_ref[...],
            (((0,), (0,)), ((), ())),
            preferred_element_type=jnp.float32,
        )
        + b_ref[...]
    )


_EB = 12800

_ea_mm = pl.pallas_call(
    _ea_mm_body,
    grid=(E // _EB,),
    in_specs=[
        pl.BlockSpec((DE, _EB), lambda i: (0, i)),
        pl.BlockSpec((DE, DOUT), lambda i: (0, 0)),
        pl.BlockSpec((1, DOUT), lambda i: (0, 0)),
    ],
    out_specs=pl.BlockSpec((_EB, DOUT), lambda i: (i, 0)),
    out_shape=jax.ShapeDtypeStruct((E, DOUT), jnp.float32),
)


def _sc_gather_body(y1_hbm, y2_hbm, ea_hbm, row_hbm, col_hbm, out_hbm,
                    row_v, col_v,
                    a1, a2, ae, b1, b2, be, oa, ob,
                    sga, sgb, swa, swb):
    wid = lax.axis_index("s") * NC + lax.axis_index("c")
    base = wid * EW
    pltpu.sync_copy(row_hbm.at[pl.ds(base, EW)], row_v)
    pltpu.sync_copy(col_hbm.at[pl.ds(base, EW)], col_v)

    def issue_gathers(c, g1, g2, ge, sem):
        off = c * C
        pltpu.async_copy(y1_hbm.at[row_v.at[pl.ds(off, C)]], g1, sem)
        pltpu.async_copy(y2_hbm.at[col_v.at[pl.ds(off, C)]], g2, sem)
        pltpu.async_copy(ea_hbm.at[pl.ds(base + off, C)], ge, sem)

    def wait_gathers(c, g1, g2, ge, sem):
        off = c * C
        pltpu.make_async_copy(y1_hbm.at[row_v.at[pl.ds(off, C)]], g1, sem).wait()
        pltpu.make_async_copy(y2_hbm.at[col_v.at[pl.ds(off, C)]], g2, sem).wait()
        pltpu.make_async_copy(ea_hbm.at[pl.ds(base + off, C)], ge, sem).wait()

    def issue_write(c, o, sem):
        pltpu.async_copy(o, out_hbm.at[pl.ds(base + c * C, C)], sem)

    def wait_write(c, o, sem):
        pltpu.make_async_copy(o, out_hbm.at[pl.ds(base + c * C, C)], sem).wait()

    def compute(g1, g2, ge, o):
        def row_body(i, carry):
            for k in range(VPR):
                sl = pl.ds(k * L, L)
                o[i, sl] = g1[i, sl] + g2[i, sl] + ge[i, sl]
            return carry

        lax.fori_loop(0, C, row_body, 0)

    issue_gathers(0, a1, a2, ae, sga)
    issue_gathers(1, b1, b2, be, sgb)

    def pair_body(p, carry):
        c0 = 2 * p
        c1 = c0 + 1
        wait_gathers(c0, a1, a2, ae, sga)

        @pl.when(c0 >= 2)
        def _():
            wait_write(c0 - 2, oa, swa)

        compute(a1, a2, ae, oa)
        issue_write(c0, oa, swa)
        issue_gathers(c0 + 2, a1, a2, ae, sga)
        wait_gathers(c1, b1, b2, be, sgb)

        @pl.when(c1 >= 3)
        def _():
            wait_write(c1 - 2, ob, swb)

        compute(b1, b2, be, ob)
        issue_write(c1, ob, swb)

        @pl.when(c1 + 2 < NCHUNK)
        def _():
            issue_gathers(c1 + 2, b1, b2, be, sgb)

        return carry

    lax.fori_loop(0, NPAIR, pair_body, 0)

    cl = NCHUNK - 1
    wait_gathers(cl, a1, a2, ae, sga)
    wait_write(cl - 2, oa, swa)
    compute(a1, a2, ae, oa)
    issue_write(cl, oa, swa)
    wait_write(cl, oa, swa)
    wait_write(cl - 1, ob, swb)


_sc_gather = functools.partial(
    pl.kernel,
    out_type=jax.ShapeDtypeStruct((E, DOUT), jnp.float32),
    mesh=plsc.VectorSubcoreMesh(core_axis_name="c", subcore_axis_name="s"),
    scratch_types=[
        pltpu.VMEM((EW,), jnp.int32),
        pltpu.VMEM((EW,), jnp.int32),
        pltpu.VMEM((C, DOUT), jnp.float32),
        pltpu.VMEM((C, DOUT), jnp.float32),
        pltpu.VMEM((C, DOUT), jnp.float32),
        pltpu.VMEM((C, DOUT), jnp.float32),
        pltpu.VMEM((C, DOUT), jnp.float32),
        pltpu.VMEM((C, DOUT), jnp.float32),
        pltpu.VMEM((C, DOUT), jnp.float32),
        pltpu.VMEM((C, DOUT), jnp.float32),
        pltpu.SemaphoreType.DMA,
        pltpu.SemaphoreType.DMA,
        pltpu.SemaphoreType.DMA,
        pltpu.SemaphoreType.DMA,
    ],
)(_sc_gather_body)


def kernel(x, edge_index, edge_attr, W, b):
    w1 = W[:D]
    w2 = W[D:2 * D]
    w3 = W[2 * D:]
    row = edge_index[0]
    col = edge_index[1]
    y1, y2 = _node_mm(x, w1, w2)
    ea = _ea_mm(edge_attr.T, w3, b.reshape(1, DOUT))
    return _sc_gather(y1, y2, ea, row, col)
